# Initial kernel scaffold; baseline (speedup 1.0000x reference)
import jax
import jax.numpy as jnp
from jax.experimental import pallas as pl


def kernel(images, layer_rect, edges, bbox, labels, node_indices, extra, W_pos, b_pos, W_img, b_img, W_edge, b_edge, W1, b1, W2, b2, W_cls, b_cls, W_loc, b_loc):
    raise NotImplementedError("write your pallas kernel here")


# speedup vs baseline: 1.9545x; 1.9545x over previous
import functools

import jax
import jax.numpy as jnp
from jax import lax
from jax.experimental import pallas as pl
from jax.experimental.pallas import tpu as pltpu
from jax.experimental.pallas import tpu_sc as plsc

_N = 10000
_E = 320000
_D = 128
_NCLS = 24
_NW = 32
_GCH = 160
_GNBLK = _E // _GCH
_GNB_BASE = _GNBLK // _NW
_GNB_REM = _GNBLK - _GNB_BASE * _NW
_SCH = 128
_SNBLK = _E // _SCH
_SNB_BASE = _SNBLK // _NW
_SNB_REM = _SNBLK - _SNB_BASE * _NW
_TILES = 16
_NP = 10240
_RPT = _NP // _TILES

_CLS_W = [0.00046026, 0.0010917, 0.00030843, 0.00069754, 0.00018872,
          0.0050046, 0.0064896, 0.17365, 0.014079, 0.021542, 0.013265,
          0.06367, 0.026286, 0.020687, 0.077436, 0.14326, 0.014923,
          0.012906, 0.00212, 0.00059179, 0.01802, 0.00026179, 0.27287,
          0.1102]


def _angle_ladder(a, axis):
    s1 = jnp.sin(a)
    c1 = jnp.cos(a)
    s2 = 2.0 * s1 * c1
    c2 = 1.0 - 2.0 * s1 * s1
    s4 = 2.0 * s2 * c2
    c4 = 1.0 - 2.0 * s2 * s2
    s8 = 2.0 * s4 * c4
    c8 = 1.0 - 2.0 * s4 * s4
    return jnp.concatenate([a, s1, s2, s4, s8, c1, c2, c4, c8], axis=axis)


def _node_emb_body(rect_ref, img_ref, wp_ref, wi_ref, b_ref, x_ref):
    feat = _angle_ladder(rect_ref[...], axis=1)
    x = jnp.dot(feat, wp_ref[...], preferred_element_type=jnp.float32)
    x += jnp.dot(img_ref[...], wi_ref[...], preferred_element_type=jnp.float32)
    x_ref[...] = x + b_ref[...]


def _node_emb(rect, images, wp, wi, bsum):
    blk = 1000
    grid = _N // blk
    return pl.pallas_call(
        _node_emb_body,
        grid=(grid,),
        in_specs=[
            pl.BlockSpec((blk, 4), lambda i: (i, 0)),
            pl.BlockSpec((blk, _D), lambda i: (i, 0)),
            pl.BlockSpec((36, _D), lambda i: (0, 0)),
            pl.BlockSpec((_D, _D), lambda i: (0, 0)),
            pl.BlockSpec((1, _D), lambda i: (0, 0)),
        ],
        out_specs=pl.BlockSpec((blk, _D), lambda i: (i, 0)),
        out_shape=jax.ShapeDtypeStruct((_N, _D), jnp.float32),
    )(rect, images, wp, wi, bsum)


def _edge_gather(rpad, x, src, dst):
    mesh = plsc.VectorSubcoreMesh(core_axis_name="c", subcore_axis_name="s")

    @functools.partial(
        pl.kernel,
        out_type=[jax.ShapeDtypeStruct((_E, 16), jnp.float32),
                  jax.ShapeDtypeStruct((_E, _D), jnp.float32)],
        mesh=mesh,
        scratch_types=[
            pltpu.VMEM((_GCH,), jnp.int32),
            pltpu.VMEM((_GCH,), jnp.int32),
            pltpu.VMEM((_GCH, _D), jnp.float32),
            pltpu.VMEM((_GCH, _D), jnp.float32),
            pltpu.VMEM((_GCH, _D), jnp.float32),
            pltpu.VMEM((_GCH, 16), jnp.float32),
        ],
    )
    def k(rpad_hbm, x_hbm, src_hbm, dst_hbm, ab_hbm, xs_hbm,
          si_v, di_v, rs_v, rd_v, xs_v, o_v):
        cid = lax.axis_index("c")
        sid = lax.axis_index("s")
        wid = sid * 2 + cid
        nb = _GNB_BASE + jnp.where(wid < _GNB_REM, 1, 0)
        sl16 = pl.ds(0, 16)

        def body(j, carry):
            blk = wid + j * _NW
            base = blk * _GCH
            pltpu.sync_copy(src_hbm.at[pl.ds(base, _GCH)], si_v)
            pltpu.sync_copy(dst_hbm.at[pl.ds(base, _GCH)], di_v)
            pltpu.sync_copy(rpad_hbm.at[si_v], rs_v)
            pltpu.sync_copy(rpad_hbm.at[di_v], rd_v)
            pltpu.sync_copy(x_hbm.at[si_v], xs_v)
            for e in range(_GCH):
                o_v[e, sl16] = jnp.abs(rs_v[e, sl16] - rd_v[e, sl16])
            pltpu.sync_copy(o_v, ab_hbm.at[pl.ds(base, _GCH)])
            pltpu.sync_copy(xs_v, xs_hbm.at[pl.ds(base, _GCH)])
            return carry

        lax.fori_loop(0, nb, body, 0)

    return k(rpad, x, src, dst)


def _edge_msg_body(ab_ref, xs_ref, we_ref, b_ref, m_ref):
    feat = _angle_ladder(ab_ref[:, :4], axis=1)
    ep = jnp.dot(feat, we_ref[...], preferred_element_type=jnp.float32)
    m_ref[...] = jnp.maximum(xs_ref[...] + ep + b_ref[...], 0.0)


def _edge_msg(ab, xs, we, be):
    blk = 4000
    grid = _E // blk
    return pl.pallas_call(
        _edge_msg_body,
        grid=(grid,),
        in_specs=[
            pl.BlockSpec((blk, 16), lambda i: (i, 0)),
            pl.BlockSpec((blk, _D), lambda i: (i, 0)),
            pl.BlockSpec((36, _D), lambda i: (0, 0)),
            pl.BlockSpec((1, _D), lambda i: (0, 0)),
        ],
        out_specs=pl.BlockSpec((blk, _D), lambda i: (i, 0)),
        out_shape=jax.ShapeDtypeStruct((_E, _D), jnp.float32),
    )(ab, xs, we, be)


def _scatter_add(m, dst, zrows):
    mesh = plsc.VectorSubcoreMesh(core_axis_name="c", subcore_axis_name="s")

    @functools.partial(
        pl.kernel,
        out_type=[jax.ShapeDtypeStruct((_NP, _D), jnp.float32),
                  jax.ShapeDtypeStruct((_NP, _D), jnp.float32)],
        mesh=mesh,
        scratch_types=[
            pltpu.VMEM((_SCH,), jnp.int32),
            pltpu.VMEM((_SCH, _D), jnp.float32),
            pltpu.VMEM_SHARED((_NP, _D), jnp.float32),
        ],
    )
    def k(m_hbm, dst_hbm, z_hbm, out0, out1, di_v, m_v, acc):
        cid = lax.axis_index("c")
        sid = lax.axis_index("s")
        wid = sid * 2 + cid
        rows = pl.ds(sid * _RPT, _RPT)
        pltpu.sync_copy(z_hbm, acc.at[rows])
        plsc.subcore_barrier()
        nb = _SNB_BASE + jnp.where(wid < _SNB_REM, 1, 0)

        def body(j, carry):
            blk = wid + j * _NW
            base = blk * _SCH
            pltpu.sync_copy(dst_hbm.at[pl.ds(base, _SCH)], di_v)
            pltpu.sync_copy(m_hbm.at[pl.ds(base, _SCH)], m_v)
            pltpu.sync_copy(m_v, acc.at[di_v], add=True)
            return carry

        lax.fori_loop(0, nb, body, 0)
        plsc.subcore_barrier()

        @pl.when(cid == 0)
        def _():
            pltpu.sync_copy(acc.at[rows], out0.at[rows])

        @pl.when(cid == 1)
        def _():
            pltpu.sync_copy(acc.at[rows], out1.at[rows])

    return k(m, dst, zrows)


def _head_body(x_ref, p0_ref, p1_ref, lbl_ref, bbox_ref, w1_ref, b1_ref,
               w2_ref, b2_ref, wc_ref, bc_ref, wl_ref, bl_ref, cw_ref,
               logits_ref, bb_ref, acc_ref):
    i = pl.program_id(0)
    h = x_ref[...] + p0_ref[...] + p1_ref[...]
    z = jnp.maximum(
        jnp.dot(h, w1_ref[...], preferred_element_type=jnp.float32)
        + b1_ref[...], 0.0)
    g = jnp.dot(z, w2_ref[...], preferred_element_type=jnp.float32) + b2_ref[...]
    logits = jnp.dot(g, wc_ref[...], preferred_element_type=jnp.float32) + bc_ref[...]
    bb = jnp.dot(g, wl_ref[...], preferred_element_type=jnp.float32) + bl_ref[...]
    logits_ref[...] = logits
    bb_ref[...] = bb
    mx = jnp.max(logits, axis=1, keepdims=True)
    lse = mx + jnp.log(jnp.sum(jnp.exp(logits - mx), axis=1, keepdims=True))
    logp = logits - lse
    lane = lax.broadcasted_iota(jnp.int32, logits.shape, 1)
    oh = (lbl_ref[...] == lane).astype(jnp.float32)
    nll = -jnp.sum(logp * oh, axis=1)
    w = jnp.sum(cw_ref[...] * oh, axis=1)
    d = bb - bbox_ref[...]
    ad = jnp.abs(d)
    sl1 = jnp.where(ad < 1.0, 0.5 * d * d, ad - 0.5)
    l128 = lax.broadcasted_iota(jnp.int32, (1, _D), 1)
    vals = (jnp.where(l128 == 0, jnp.sum(w * nll), 0.0)
            + jnp.where(l128 == 1, jnp.sum(w), 0.0)
            + jnp.where(l128 == 2, jnp.sum(sl1), 0.0))

    @pl.when(i == 0)
    def _():
        acc_ref[...] = vals

    @pl.when(i != 0)
    def _():
        acc_ref[...] = acc_ref[...] + vals


def _head(x, p0, p1, labels2, bbox, w1, b1, w2, b2, wc, bc, wl, bl, cw):
    blk = 1000
    grid = _N // blk
    nd = pl.BlockSpec((blk, _D), lambda i: (i, 0))
    return pl.pallas_call(
        _head_body,
        grid=(grid,),
        in_specs=[
            nd, nd, nd,
            pl.BlockSpec((blk, 1), lambda i: (i, 0)),
            pl.BlockSpec((blk, 4), lambda i: (i, 0)),
            pl.BlockSpec((_D, _D), lambda i: (0, 0)),
            pl.BlockSpec((1, _D), lambda i: (0, 0)),
            pl.BlockSpec((_D, _D), lambda i: (0, 0)),
            pl.BlockSpec((1, _D), lambda i: (0, 0)),
            pl.BlockSpec((_D, _NCLS), lambda i: (0, 0)),
            pl.BlockSpec((1, _NCLS), lambda i: (0, 0)),
            pl.BlockSpec((_D, 4), lambda i: (0, 0)),
            pl.BlockSpec((1, 4), lambda i: (0, 0)),
            pl.BlockSpec((1, _NCLS), lambda i: (0, 0)),
        ],
        out_specs=[
            pl.BlockSpec((blk, _NCLS), lambda i: (i, 0)),
            pl.BlockSpec((blk, 4), lambda i: (i, 0)),
            pl.BlockSpec((1, _D), lambda i: (0, 0)),
        ],
        out_shape=[
            jax.ShapeDtypeStruct((_N, _NCLS), jnp.float32),
            jax.ShapeDtypeStruct((_N, 4), jnp.float32),
            jax.ShapeDtypeStruct((1, _D), jnp.float32),
        ],
    )(x, p0, p1, labels2, bbox, w1, b1, w2, b2, wc, bc, wl, bl, cw)


def kernel(images, layer_rect, edges, bbox, labels, node_indices, extra,
           W_pos, b_pos, W_img, b_img, W_edge, b_edge, W1, b1, W2, b2,
           W_cls, b_cls, W_loc, b_loc):
    f32 = jnp.float32
    src = edges[0]
    dst = edges[1]
    perm = jnp.array(
        [0, 1, 2, 3]
        + [4 + i * 4 + kk for kk in range(4) for i in range(4)]
        + [20 + i * 4 + kk for kk in range(4) for i in range(4)],
        dtype=jnp.int32)
    wp = W_pos[perm]
    we = W_edge[perm]
    bsum = (b_pos + b_img).reshape(1, _D)
    rpad = jnp.pad(layer_rect, ((0, 0), (0, _D - 4)))

    x = _node_emb(layer_rect, images, wp, W_img, bsum)
    ab, xs = _edge_gather(rpad, x, src, dst)
    m = _edge_msg(ab, xs, we, b_edge.reshape(1, _D))
    zrows = jnp.zeros((_RPT, _D), f32)
    p0, p1 = _scatter_add(m, dst, zrows)
    p0 = p0[:_N]
    p1 = p1[:_N]
    cw = jnp.array(_CLS_W, f32).reshape(1, _NCLS)
    logits, bb, acc = _head(x, p0, p1, labels.reshape(_N, 1), bbox,
                            W1, b1.reshape(1, _D), W2, b2.reshape(1, _D),
                            W_cls, b_cls.reshape(1, _NCLS),
                            W_loc, b_loc.reshape(1, 4), cw)
    loss = acc[0, 0] / acc[0, 1] + acc[0, 2] / (4.0 * _N)
    return (logits, bb, loss.astype(f32).reshape(()))
